# Initial kernel scaffold; baseline (speedup 1.0000x reference)
import jax
import jax.numpy as jnp
from jax.experimental import pallas as pl


def kernel(x_paper, x_author, edge_index_paper__to__author, edge_index_author__to__paper, edge_index_paper__cites__paper, l0_proj_paper_W, l0_proj_paper_b, l0_proj_author_W, l0_proj_author_b, l0_att_src_paper__to__author, l0_att_dst_paper__to__author, l0_att_src_author__to__paper, l0_att_dst_author__to__paper, l0_att_src_paper__cites__paper, l0_att_dst_paper__cites__paper, l0_q, l0_klin_W, l0_klin_b, l1_proj_paper_W, l1_proj_paper_b, l1_proj_author_W, l1_proj_author_b, l1_att_src_paper__to__author, l1_att_dst_paper__to__author, l1_att_src_author__to__paper, l1_att_dst_author__to__paper, l1_att_src_paper__cites__paper, l1_att_dst_paper__cites__paper, l1_q, l1_klin_W, l1_klin_b):
    raise NotImplementedError("write your pallas kernel here")


# speedup vs baseline: 1.8123x; 1.8123x over previous
import functools

import jax
import jax.numpy as jnp
from jax.experimental import pallas as pl

_N = 50000
_BLK = 2000
_NBLK = _N // _BLK
_CH = 128

_NODE_TYPES = ("paper", "author")
_EDGE_TYPES = (("paper", "to", "author"), ("author", "to", "paper"),
               ("paper", "cites", "paper"))


def _ename(et):
    return et[0] + "__" + et[1] + "__" + et[2]


def _blockdiag(att):
    h, d = att.shape
    eye = jnp.eye(h, dtype=att.dtype)
    return (att[:, :, None] * eye[:, None, :]).reshape(h * d, h)


def _proj_body(x_ref, w_ref, b_ref, a_ref, xp_ref, av_ref):
    xp = jnp.dot(x_ref[...], w_ref[...],
                 preferred_element_type=jnp.float32) + b_ref[...]
    xp_ref[...] = xp
    av_ref[...] = jnp.dot(xp, a_ref[...], preferred_element_type=jnp.float32)


def _proj(x, W, b, A):
    return pl.pallas_call(
        _proj_body,
        grid=(_NBLK,),
        in_specs=[
            pl.BlockSpec((_BLK, _CH), lambda i: (i, 0)),
            pl.BlockSpec((_CH, _CH), lambda i: (0, 0)),
            pl.BlockSpec((1, _CH), lambda i: (0, 0)),
            pl.BlockSpec((_CH, _CH), lambda i: (0, 0)),
        ],
        out_specs=[
            pl.BlockSpec((_BLK, _CH), lambda i: (i, 0)),
            pl.BlockSpec((_BLK, _CH), lambda i: (i, 0)),
        ],
        out_shape=[
            jax.ShapeDtypeStruct((_N, _CH), jnp.float32),
            jax.ShapeDtypeStruct((_N, _CH), jnp.float32),
        ],
    )(x, W, b, A)


def _kmat_body(s_ref, w_ref, b_ref, o_ref):
    i = pl.program_id(1)
    t = jnp.tanh(jnp.dot(s_ref[0], w_ref[...],
                         preferred_element_type=jnp.float32) + b_ref[...])
    part = t.sum(axis=0, keepdims=True).reshape(1, 1, _CH)

    @pl.when(i == 0)
    def _():
        o_ref[...] = part

    @pl.when(i > 0)
    def _():
        o_ref[...] = o_ref[...] + part


def _kmat_sum(stacked, W, b):
    K = stacked.shape[0]
    return pl.pallas_call(
        _kmat_body,
        grid=(K, _NBLK),
        in_specs=[
            pl.BlockSpec((1, _BLK, _CH), lambda k, i: (k, i, 0)),
            pl.BlockSpec((_CH, _CH), lambda k, i: (0, 0)),
            pl.BlockSpec((1, _CH), lambda k, i: (0, 0)),
        ],
        out_specs=pl.BlockSpec((1, 1, _CH), lambda k, i: (k, 0, 0)),
        out_shape=jax.ShapeDtypeStruct((K, 1, _CH), jnp.float32),
    )(stacked, W, b).reshape(K, _CH)


def _combine_body(s_ref, a_ref, o_ref, *, mode):
    K = s_ref.shape[0]
    res = jnp.zeros(o_ref.shape, jnp.float32)
    for k in range(K):
        res = res + a_ref[0, k] * s_ref[k]
    if mode == "elu":
        res = jnp.where(res > 0, res, jnp.exp(res) - 1.0)
    elif mode == "l2norm":
        nrm = jnp.sqrt(jnp.sum(res * res, axis=1, keepdims=True))
        res = res / jnp.maximum(nrm, 1e-12)
    o_ref[...] = res


def _combine(stacked, attn, mode):
    K = stacked.shape[0]
    body = functools.partial(_combine_body, mode=mode)
    attn_pad = jnp.zeros((1, _CH), jnp.float32).at[0, :K].set(attn)
    return pl.pallas_call(
        body,
        grid=(_NBLK,),
        in_specs=[
            pl.BlockSpec((K, _BLK, _CH), lambda i: (0, i, 0)),
            pl.BlockSpec((1, _CH), lambda i: (0, 0)),
        ],
        out_specs=pl.BlockSpec((_BLK, _CH), lambda i: (i, 0)),
        out_shape=jax.ShapeDtypeStruct((_N, _CH), jnp.float32),
    )(stacked, attn_pad)


def _segment_softmax(alpha, seg, num_seg):
    amax = jax.ops.segment_max(alpha, seg, num_segments=num_seg)
    amax = jnp.where(jnp.isfinite(amax), amax, 0.0)
    ex = jnp.exp(alpha - amax[seg])
    den = jax.ops.segment_sum(ex, seg, num_segments=num_seg)
    return ex / (den[seg] + 1e-16)


def _han_layer(x_dict, edges, p, l, heads, out_ch):
    d = out_ch // heads
    xp = {}
    av = {}
    cols = {nt: [] for nt in _NODE_TYPES}
    for et in _EDGE_TYPES:
        src, _, dst = et
        en = _ename(et)
        cols[src].append((en, "src"))
        cols[dst].append((en, "dst"))
    for nt in _NODE_TYPES:
        mats = [_blockdiag(p[f"l{l}_att_{role}_{en}"]) for en, role in cols[nt]]
        A = jnp.concatenate(mats, axis=1)
        ncols = A.shape[1]
        A = jnp.pad(A, ((0, 0), (0, _CH - ncols)))
        xp_nt, av_nt = _proj(x_dict[nt], p[f"l{l}_proj_{nt}_W"],
                             p[f"l{l}_proj_{nt}_b"].reshape(1, _CH), A)
        xp[nt] = xp_nt
        avd = {}
        off = 0
        for en, role in cols[nt]:
            avd[(en, role)] = av_nt[:, off:off + heads]
            off += heads
        av[nt] = avd

    acc = {nt: [] for nt in _NODE_TYPES}
    for et in _EDGE_TYPES:
        src, _, dst = et
        en = _ename(et)
        ei = edges["edge_index_" + en]
        row, col = ei[0], ei[1]
        a_src = av[src][(en, "src")]
        a_dst = av[dst][(en, "dst")]
        alpha = a_src[row] + a_dst[col]
        alpha = jnp.where(alpha >= 0, alpha, 0.2 * alpha)
        alpha = _segment_softmax(alpha, col, _N)
        if heads == 1:
            msg = xp[src][row] * alpha
        else:
            msg = (xp[src][row].reshape(-1, heads, d)
                   * alpha[:, :, None]).reshape(-1, out_ch)
        out = jax.ops.segment_sum(msg, col, num_segments=_N)
        acc[dst].append(jax.nn.relu(out))

    res = {}
    for nt in _NODE_TYPES:
        stacked = jnp.stack(acc[nt])
        K = stacked.shape[0]
        ksum = _kmat_sum(stacked, p[f"l{l}_klin_W"],
                         p[f"l{l}_klin_b"].reshape(1, _CH))
        kmat = ksum / _N
        score = kmat @ p[f"l{l}_q"]
        attn = jax.nn.softmax(score, axis=0)
        mode = "elu" if l == 0 else "l2norm"
        res[nt] = _combine(stacked, attn, mode)
    return res


def kernel(x_paper, x_author, edge_index_paper__to__author, edge_index_author__to__paper, edge_index_paper__cites__paper, l0_proj_paper_W, l0_proj_paper_b, l0_proj_author_W, l0_proj_author_b, l0_att_src_paper__to__author, l0_att_dst_paper__to__author, l0_att_src_author__to__paper, l0_att_dst_author__to__paper, l0_att_src_paper__cites__paper, l0_att_dst_paper__cites__paper, l0_q, l0_klin_W, l0_klin_b, l1_proj_paper_W, l1_proj_paper_b, l1_proj_author_W, l1_proj_author_b, l1_att_src_paper__to__author, l1_att_dst_paper__to__author, l1_att_src_author__to__paper, l1_att_dst_author__to__paper, l1_att_src_paper__cites__paper, l1_att_dst_paper__cites__paper, l1_q, l1_klin_W, l1_klin_b):
    kw = dict(locals())
    edges = {k: v for k, v in kw.items() if k.startswith("edge_index_")}
    floats = {k: v for k, v in kw.items() if not k.startswith("edge_index_")}
    h = {"paper": x_paper, "author": x_author}
    h = _han_layer(h, edges, floats, 0, 8, 128)
    h = _han_layer(h, edges, floats, 1, 1, 128)
    return (h["paper"], h["author"])
